# baseline (speedup 1.0000x reference)
import functools

import jax
import jax.numpy as jnp
from jax import lax
from jax.experimental import pallas as pl
from jax.experimental.pallas import tpu as pltpu
from jax.experimental.pallas import tpu_sc as plsc

NC, NS, LN = 2, 16, 16
CH = 32
CH3 = 64
CB = 96
FB = 32
_SC_PARAMS = pltpu.CompilerParams(needs_layout_passes=False)


def _phase0_matmuls(xp, Wl4, Wr4, NP, H, D):
    BN = 1024

    def body(x_ref, wl_ref, wr_ref, ol_ref, or_ref):
        xb = x_ref[...]
        ol_ref[0] = jnp.dot(xb, wl_ref[0], preferred_element_type=jnp.float32)
        or_ref[0] = jnp.dot(xb, wr_ref[0], preferred_element_type=jnp.float32)

    return pl.pallas_call(
        body,
        grid=(H, NP // BN),
        in_specs=[
            pl.BlockSpec((BN, D), lambda h, i: (i, 0)),
            pl.BlockSpec((1, D, D), lambda h, i: (h, 0, 0)),
            pl.BlockSpec((1, D, D), lambda h, i: (h, 0, 0)),
        ],
        out_specs=[
            pl.BlockSpec((1, BN, D), lambda h, i: (h, i, 0)),
            pl.BlockSpec((1, BN, D), lambda h, i: (h, i, 0)),
        ],
        out_shape=[jax.ShapeDtypeStruct((H, NP, D), jnp.float32)] * 2,
    )(xp, Wl4, Wr4)


def _phase1_gat(xlt, xrt, srcf, dstf, attf, NP, EP, H, D):
    esc = EP // NS
    n_chunks = esc // CH
    rows_sc = NP // NS
    mesh = plsc.VectorSubcoreMesh(
        core_axis_name="c", subcore_axis_name="s", num_cores=NC, num_subcores=NS
    )
    idx_t = [pltpu.VMEM((CH,), jnp.int32)] * 5
    set_t = idx_t + [pltpu.VMEM((CH, D), jnp.float32)] * 2

    @functools.partial(
        pl.kernel,
        mesh=mesh,
        out_type=[
            jax.ShapeDtypeStruct((H * NP, D), jnp.float32),
            jax.ShapeDtypeStruct((NP, D), jnp.float32),
            jax.ShapeDtypeStruct((NC * NS, NP), jnp.float32),
            jax.ShapeDtypeStruct((NS, NP), jnp.float32),
        ],
        scratch_types=(
            set_t * 3
            + [
                pltpu.VMEM((CB,), jnp.int32),
                pltpu.VMEM((D,), jnp.float32),
                pltpu.VMEM((NP,), jnp.float32),
                pltpu.VMEM((NS, D), jnp.float32),
                pltpu.VMEM((NP // NS,), jnp.float32),
                pltpu.VMEM_SHARED((NP, D), jnp.float32),
            ]
            + [pltpu.SemaphoreType.DMA] * 9
        ),
        compiler_params=_SC_PARAMS,
    )
    def k(xl_h, xr_h, src_h, dst_h, att_h, msg_out, cnt_out, dstg, cstg,
          *refs):
        S = [refs[7 * x: 7 * x + 7] for x in range(3)]
        cb_v, att_v, den_t, stg_v, red_v, acc = refs[21:27]
        gsem = [refs[27 + 2 * x: 29 + 2 * x] for x in range(3)]
        isem = [refs[33 + x] for x in range(3)]
        xr_a = S[0][6]
        cid = lax.axis_index("c")
        sid = lax.axis_index("s")
        zv = jnp.zeros((LN,), jnp.float32)
        ones = jnp.full((LN,), 1.0, jnp.float32)
        m0 = lax.iota(jnp.int32, LN) == 0

        def zero_den():
            @pl.loop(0, NP // LN)
            def _(i):
                o = pl.multiple_of(i * LN, LN)
                den_t[pl.ds(o, LN)] = zv

        def reduce_stage(stg):
            @pl.loop(0, rows_sc // D)
            def _(t):
                tD = pl.multiple_of(t * D, D)
                pltpu.sync_copy(
                    stg.at[:, pl.ds(sid * rows_sc + tD, D)], stg_v
                )

                @pl.loop(0, D // LN)
                def _(i):
                    o = pl.multiple_of(i * LN, LN)
                    sl = pl.ds(o, LN)
                    tv = zv
                    for s in range(NS):
                        tv = tv + stg_v[s, sl]
                    red_v[pl.ds(tD + o, LN)] = tv

        @pl.when(cid == 0)
        def _():
            zero_den()

            @pl.loop(0, esc // CB)
            def _(q):
                pltpu.sync_copy(dst_h.at[pl.ds(sid * esc + q * CB, CB)], cb_v)

                @pl.loop(0, CB // LN)
                def _(i):
                    o = pl.multiple_of(i * LN, LN)
                    dstv = cb_v[pl.ds(o, LN)]
                    for j in range(LN):
                        didx = jnp.full((LN,), dstv[j], jnp.int32)
                        plsc.addupdate_scatter(den_t, [didx], ones, mask=m0)

            pltpu.sync_copy(den_t, cstg.at[sid])
            plsc.subcore_barrier()
            reduce_stage(cstg)

            @pl.loop(0, rows_sc // FB)
            def _(kblk):
                kF = pl.multiple_of(kblk * FB, FB)

                @pl.loop(0, FB // LN)
                def _(i):
                    o = pl.multiple_of(i * LN, LN)
                    redv = red_v[pl.ds(kF + o, LN)]
                    for j in range(LN):
                        cv = jnp.full((LN,), redv[j])
                        for c in range(D // LN):
                            xr_a[o + j, pl.ds(c * LN, LN)] = cv

                pltpu.sync_copy(
                    xr_a.at[pl.ds(0, FB)],
                    cnt_out.at[pl.ds(sid * rows_sc + kF, FB)],
                )

        def idx_load(c, x):
            srcb, dstb = S[x][0], S[x][1]
            base = sid * esc + c * CH
            pltpu.async_copy(src_h.at[pl.ds(base, CH)], srcb, isem[x])
            pltpu.async_copy(dst_h.at[pl.ds(base, CH)], dstb, isem[x])

        def idx_wait(x):
            srcb, dstb = S[x][0], S[x][1]
            pltpu.make_async_copy(src_h.at[pl.ds(0, CH)], srcb, isem[x]).wait()
            pltpu.make_async_copy(dst_h.at[pl.ds(0, CH)], dstb, isem[x]).wait()

        def transform_gather(off, x):
            srcb, dstb, idxsb, idxdb, sdstb, xlb, xrb = S[x]

            @pl.loop(0, CH // LN)
            def _(i):
                o = pl.multiple_of(i * LN, LN)
                dv = dstb[pl.ds(o, LN)]
                idxsb[pl.ds(o, LN)] = srcb[pl.ds(o, LN)] + off
                idxdb[pl.ds(o, LN)] = dv + off
                sdstb[pl.ds(o, LN)] = dv

            pltpu.async_copy(xl_h.at[idxsb], xlb, gsem[x][0])
            pltpu.async_copy(xr_h.at[idxdb], xrb, gsem[x][1])

        def compute_chunk(x):
            _, _, idxsb, idxdb, sdstb, xlb, xrb = S[x]
            pltpu.make_async_copy(xl_h.at[idxsb], xlb, gsem[x][0]).wait()
            pltpu.make_async_copy(xr_h.at[idxdb], xrb, gsem[x][1]).wait()

            @pl.loop(0, CH // LN)
            def _(i):
                o = pl.multiple_of(i * LN, LN)
                dstv = sdstb[pl.ds(o, LN)]
                for j in range(LN):
                    e = o + j
                    accv = zv
                    for c in range(D // LN):
                        sl = pl.ds(c * LN, LN)
                        z = xlb[e, sl] + xrb[e, sl]
                        z = jnp.maximum(z, 0.2 * z)
                        accv = accv + z * att_v[sl]
                    logit = jnp.sum(accv)
                    exv = jnp.exp(jnp.full((LN,), logit))
                    for c in range(D // LN):
                        sl = pl.ds(c * LN, LN)
                        xlb[e, sl] = exv * xlb[e, sl]
                    didx = jnp.full((LN,), dstv[j], jnp.int32)
                    plsc.addupdate_scatter(den_t, [didx], exv, mask=m0)

            pltpu.sync_copy(xlb, acc.at[sdstb], add=True)

        for p in range(H // NC):
            h = cid * (H // NC) + p
            off = h * NP

            @pl.loop(0, FB)
            def _(r):
                for c in range(D // LN):
                    xr_a[r, pl.ds(c * LN, LN)] = zv

            @pl.loop(0, rows_sc // FB)
            def _(kblk):
                pltpu.sync_copy(
                    xr_a.at[pl.ds(0, FB)],
                    acc.at[pl.ds(sid * rows_sc + kblk * FB, FB)],
                )

            zero_den()
            pltpu.sync_copy(att_h.at[pl.ds(h * D, D)], att_v)
            plsc.subcore_barrier()

            idx_load(0, 0)
            idx_load(1, 1)
            idx_load(2, 2)
            idx_wait(0)
            transform_gather(off, 0)
            idx_wait(1)
            transform_gather(off, 1)

            @pl.loop(0, n_chunks // 3)
            def _(t):
                c0 = t * 3
                for x in range(3):
                    c = c0 + x
                    compute_chunk(x)

                    @pl.when(c + 3 < n_chunks)
                    def _():
                        idx_load(c + 3, x)

                    @pl.when(c + 2 < n_chunks)
                    def _():
                        z = (x + 2) % 3
                        idx_wait(z)
                        transform_gather(off, z)

            plsc.subcore_barrier()
            pltpu.sync_copy(den_t, dstg.at[cid * NS + sid])
            plsc.subcore_barrier()
            reduce_stage(dstg.at[pl.ds(cid * NS, NS)])

            @pl.loop(0, rows_sc // FB)
            def _(kblk):
                kF = pl.multiple_of(kblk * FB, FB)
                r0 = sid * rows_sc + kF
                pltpu.sync_copy(acc.at[pl.ds(r0, FB)], xr_a.at[pl.ds(0, FB)])

                @pl.loop(0, FB // LN)
                def _(i):
                    o = pl.multiple_of(i * LN, LN)
                    redv = red_v[pl.ds(kF + o, LN)]
                    for j in range(LN):
                        dv = jnp.full((LN,), redv[j] + 1e-16)
                        for c in range(D // LN):
                            sl = pl.ds(c * LN, LN)
                            xr_a[o + j, sl] = xr_a[o + j, sl] / dv

                pltpu.sync_copy(
                    xr_a.at[pl.ds(0, FB)], msg_out.at[pl.ds(off + r0, FB)]
                )

            plsc.subcore_barrier()

    return k(xlt, xrt, srcf, dstf, attf)


def _phase2_combine(nm, cnt, Wg, NP, H, D):
    BN = 512

    def body(a_ref, c_ref, wg_ref, o_ref):
        a = a_ref[...]
        og = jnp.mean(a, axis=0)
        hh = jnp.where(og > 0, og, jnp.exp(og) - 1.0)
        deg = c_ref[:, 0:1]
        dis = jnp.where(deg > 0, lax.rsqrt(deg), 0.0)
        o_ref[...] = jnp.dot(hh, wg_ref[...], preferred_element_type=jnp.float32) * dis

    return pl.pallas_call(
        body,
        grid=(NP // BN,),
        in_specs=[
            pl.BlockSpec((H, BN, D), lambda i: (0, i, 0)),
            pl.BlockSpec((BN, D), lambda i: (i, 0)),
            pl.BlockSpec((D, D), lambda i: (0, 0)),
        ],
        out_specs=pl.BlockSpec((BN, D), lambda i: (i, 0)),
        out_shape=jax.ShapeDtypeStruct((NP, D), jnp.float32),
    )(nm, cnt, Wg)


def _phase3_gcn(hgp, srcf, dstf, NP, EP, D):
    esc = EP // (NC * NS)
    n_chunks = esc // CH3
    rows_sc = NP // NS
    mesh = plsc.VectorSubcoreMesh(
        core_axis_name="c", subcore_axis_name="s", num_cores=NC, num_subcores=NS
    )

    @functools.partial(
        pl.kernel,
        mesh=mesh,
        out_type=jax.ShapeDtypeStruct((NC * NP, D), jnp.float32),
        scratch_types=[
            pltpu.VMEM((CH3,), jnp.int32),
            pltpu.VMEM((CH3,), jnp.int32),
            pltpu.VMEM((CH3,), jnp.int32),
            pltpu.VMEM((CH3,), jnp.int32),
            pltpu.VMEM((CH3, D), jnp.float32),
            pltpu.VMEM((CH3, D), jnp.float32),
            pltpu.VMEM_SHARED((NP, D), jnp.float32),
            pltpu.SemaphoreType.DMA,
            pltpu.SemaphoreType.DMA,
        ],
        compiler_params=_SC_PARAMS,
    )
    def k(hg_h, src_h, dst_h, out_h,
          src_a, src_b, dst_a, dst_b, rows_a, rows_b, acc2, sa, sb):
        cid = lax.axis_index("c")
        sid = lax.axis_index("s")
        zv = jnp.zeros((LN,), jnp.float32)

        @pl.loop(0, FB)
        def _(r):
            for c in range(D // LN):
                rows_a[r, pl.ds(c * LN, LN)] = zv

        @pl.loop(0, rows_sc // FB)
        def _(kblk):
            pltpu.sync_copy(
                rows_a.at[pl.ds(0, FB)],
                acc2.at[pl.ds(sid * rows_sc + kblk * FB, FB)],
            )
        plsc.subcore_barrier()

        def load_chunk(c, srcb, dstb, rowsb, sg):
            base = (cid * NS + sid) * esc + c * CH3
            pltpu.sync_copy(src_h.at[pl.ds(base, CH3)], srcb)
            pltpu.sync_copy(dst_h.at[pl.ds(base, CH3)], dstb)
            pltpu.async_copy(hg_h.at[srcb], rowsb, sg)

        def flush_chunk(srcb, dstb, rowsb, sg):
            pltpu.make_async_copy(hg_h.at[srcb], rowsb, sg).wait()
            pltpu.sync_copy(rowsb, acc2.at[dstb], add=True)

        load_chunk(0, src_a, dst_a, rows_a, sa)

        @pl.loop(0, n_chunks // 2)
        def _(t):
            c0 = t * 2
            load_chunk(c0 + 1, src_b, dst_b, rows_b, sb)
            flush_chunk(src_a, dst_a, rows_a, sa)

            @pl.when(c0 + 2 < n_chunks)
            def _():
                load_chunk(c0 + 2, src_a, dst_a, rows_a, sa)

            flush_chunk(src_b, dst_b, rows_b, sb)

        plsc.subcore_barrier()

        @pl.loop(0, rows_sc // FB)
        def _(kblk):
            r0 = sid * rows_sc + kblk * FB
            pltpu.sync_copy(
                acc2.at[pl.ds(r0, FB)], out_h.at[pl.ds(cid * NP + r0, FB)]
            )
        plsc.subcore_barrier()

    return k(hgp, srcf, dstf)


def _phase4_finish(acc2, cnt, NP, D):
    BN = 512

    def body(a2_ref, c_ref, o_ref):
        s = a2_ref[0] + a2_ref[1]
        deg = c_ref[:, 0:1]
        dis = jnp.where(deg > 0, lax.rsqrt(deg), 0.0)
        o_ref[...] = s * dis

    return pl.pallas_call(
        body,
        grid=(NP // BN,),
        in_specs=[
            pl.BlockSpec((2, BN, D), lambda i: (0, i, 0)),
            pl.BlockSpec((BN, D), lambda i: (i, 0)),
        ],
        out_specs=pl.BlockSpec((BN, D), lambda i: (i, 0)),
        out_shape=jax.ShapeDtypeStruct((NP, D), jnp.float32),
    )(acc2, cnt)


def kernel(x, edge_index, Wl, Wr, att, Wg):
    N, D = x.shape
    H = att.shape[0]
    E = edge_index.shape[1]
    NP = -(-N // 2048) * 2048
    EP1 = -(-E // (3 * NS * CH)) * (3 * NS * CH)
    EP3 = -(-E // (2 * NC * NS * CH3)) * (2 * NC * NS * CH3)
    EP = max(EP1, EP3)

    src = edge_index[0].astype(jnp.int32)
    dst = edge_index[1].astype(jnp.int32)
    pad = jnp.full((EP - E,), NP - 1, jnp.int32)
    srcf = jnp.concatenate([src, pad])
    dstf = jnp.concatenate([dst, pad])

    xp = jnp.zeros((NP, D), jnp.float32).at[:N].set(x)
    Wl4 = Wl.reshape(D, H, D).transpose(1, 0, 2)
    Wr4 = Wr.reshape(D, H, D).transpose(1, 0, 2)
    attf = att.reshape(H * D)

    xlt, xrt = _phase0_matmuls(xp, Wl4, Wr4, NP, H, D)
    xlt = xlt.reshape(H * NP, D)
    xrt = xrt.reshape(H * NP, D)

    nmf, cnt, _, _ = _phase1_gat(xlt, xrt, srcf, dstf, attf, NP, EP1, H, D)
    nm = nmf.reshape(H, NP, D)

    hgp = _phase2_combine(nm, cnt, Wg, NP, H, D)

    acc2f = _phase3_gcn(hgp, srcf, dstf, NP, EP3, D)
    acc2 = acc2f.reshape(NC, NP, D)

    out = _phase4_finish(acc2, cnt, NP, D)
    return out[:N]


# speedup vs baseline: 1.2087x; 1.2087x over previous
import functools

import jax
import jax.numpy as jnp
from jax import lax
from jax.experimental import pallas as pl
from jax.experimental.pallas import tpu as pltpu
from jax.experimental.pallas import tpu_sc as plsc

NC, NS, LN = 2, 16, 16
CH = 64
CH3 = 128
CB = 256
FB = 64
_SC_PARAMS = pltpu.CompilerParams(needs_layout_passes=False)


def _phase0_matmuls(xp, Wl4, Wr4, NP, H, D):
    BN = 1024

    def body(x_ref, wl_ref, wr_ref, ol_ref, or_ref):
        xb = x_ref[...]
        ol_ref[0] = jnp.dot(xb, wl_ref[0], preferred_element_type=jnp.float32)
        or_ref[0] = jnp.dot(xb, wr_ref[0], preferred_element_type=jnp.float32)

    return pl.pallas_call(
        body,
        grid=(H, NP // BN),
        in_specs=[
            pl.BlockSpec((BN, D), lambda h, i: (i, 0)),
            pl.BlockSpec((1, D, D), lambda h, i: (h, 0, 0)),
            pl.BlockSpec((1, D, D), lambda h, i: (h, 0, 0)),
        ],
        out_specs=[
            pl.BlockSpec((1, BN, D), lambda h, i: (h, i, 0)),
            pl.BlockSpec((1, BN, D), lambda h, i: (h, i, 0)),
        ],
        out_shape=[jax.ShapeDtypeStruct((H, NP, D), jnp.float32)] * 2,
    )(xp, Wl4, Wr4)


def _phase1_gat(xlt, xrt, srcf, dstf, attf, NP, EP, H, D):
    esc = EP // NS
    n_chunks = esc // CH
    rows_sc = NP // NS
    mesh = plsc.VectorSubcoreMesh(
        core_axis_name="c", subcore_axis_name="s", num_cores=NC, num_subcores=NS
    )

    @functools.partial(
        pl.kernel,
        mesh=mesh,
        out_type=[
            jax.ShapeDtypeStruct((H * NP, D), jnp.float32),
            jax.ShapeDtypeStruct((NP, D), jnp.float32),
            jax.ShapeDtypeStruct((NC * NS, NP), jnp.float32),
            jax.ShapeDtypeStruct((NS, NP), jnp.float32),
        ],
        scratch_types=[
            pltpu.VMEM((CH,), jnp.int32),
            pltpu.VMEM((CH,), jnp.int32),
            pltpu.VMEM((CH,), jnp.int32),
            pltpu.VMEM((CH,), jnp.int32),
            pltpu.VMEM((CH,), jnp.int32),
            pltpu.VMEM((CH,), jnp.int32),
            pltpu.VMEM((CH,), jnp.int32),
            pltpu.VMEM((CH,), jnp.int32),
            pltpu.VMEM((CH, D), jnp.float32),
            pltpu.VMEM((CH, D), jnp.float32),
            pltpu.VMEM((CH, D), jnp.float32),
            pltpu.VMEM((CH, D), jnp.float32),
            pltpu.VMEM((CH,), jnp.int32),
            pltpu.VMEM((CH,), jnp.int32),
            pltpu.VMEM((CB,), jnp.int32),
            pltpu.VMEM((D,), jnp.float32),
            pltpu.VMEM((NP,), jnp.float32),
            pltpu.VMEM((NS, D), jnp.float32),
            pltpu.VMEM((NP // NS,), jnp.float32),
            pltpu.VMEM_SHARED((NP, D), jnp.float32),
            pltpu.SemaphoreType.DMA,
            pltpu.SemaphoreType.DMA,
            pltpu.SemaphoreType.DMA,
            pltpu.SemaphoreType.DMA,
            pltpu.SemaphoreType.DMA,
            pltpu.SemaphoreType.DMA,
        ],
        compiler_params=_SC_PARAMS,
    )
    def k(xl_h, xr_h, src_h, dst_h, att_h, msg_out, cnt_out, dstg, cstg,
          src_a, src_b, dst_a, dst_b, idxs_a, idxs_b, idxd_a, idxd_b,
          xl_a, xl_b, xr_a, xr_b, sdst_a, sdst_b, cb_v, att_v,
          den_t, stg_v, red_v, acc,
          s1a, s2a, s1b, s2b, ia, ib):
        cid = lax.axis_index("c")
        sid = lax.axis_index("s")
        zv = jnp.zeros((LN,), jnp.float32)
        ones = jnp.full((LN,), 1.0, jnp.float32)
        m0 = lax.iota(jnp.int32, LN) == 0

        def zero_den():
            @pl.loop(0, NP // LN)
            def _(i):
                o = pl.multiple_of(i * LN, LN)
                den_t[pl.ds(o, LN)] = zv

        def reduce_stage(stg):
            @pl.loop(0, rows_sc // D)
            def _(t):
                tD = pl.multiple_of(t * D, D)
                pltpu.sync_copy(
                    stg.at[:, pl.ds(sid * rows_sc + tD, D)], stg_v
                )

                @pl.loop(0, D // LN)
                def _(i):
                    o = pl.multiple_of(i * LN, LN)
                    sl = pl.ds(o, LN)
                    tv = zv
                    for s in range(NS):
                        tv = tv + stg_v[s, sl]
                    red_v[pl.ds(tD + o, LN)] = tv

        @pl.when(cid == 0)
        def _():
            zero_den()

            @pl.loop(0, esc // CB)
            def _(q):
                pltpu.sync_copy(dst_h.at[pl.ds(sid * esc + q * CB, CB)], cb_v)

                @pl.loop(0, CB // LN)
                def _(i):
                    o = pl.multiple_of(i * LN, LN)
                    dstv = cb_v[pl.ds(o, LN)]
                    for j in range(LN):
                        didx = jnp.full((LN,), dstv[j], jnp.int32)
                        plsc.addupdate_scatter(den_t, [didx], ones, mask=m0)

            pltpu.sync_copy(den_t, cstg.at[sid])
            plsc.subcore_barrier()
            reduce_stage(cstg)

            @pl.loop(0, rows_sc // FB)
            def _(kblk):
                kF = pl.multiple_of(kblk * FB, FB)

                @pl.loop(0, FB // LN)
                def _(i):
                    o = pl.multiple_of(i * LN, LN)
                    redv = red_v[pl.ds(kF + o, LN)]
                    for j in range(LN):
                        cv = jnp.full((LN,), redv[j])
                        for c in range(D // LN):
                            xr_a[o + j, pl.ds(c * LN, LN)] = cv

                pltpu.sync_copy(
                    xr_a.at[pl.ds(0, FB)],
                    cnt_out.at[pl.ds(sid * rows_sc + kF, FB)],
                )

        def idx_load(c, srcb, dstb, sem):
            base = sid * esc + c * CH
            pltpu.async_copy(src_h.at[pl.ds(base, CH)], srcb, sem)
            pltpu.async_copy(dst_h.at[pl.ds(base, CH)], dstb, sem)

        def idx_wait(srcb, dstb, sem):
            pltpu.make_async_copy(src_h.at[pl.ds(0, CH)], srcb, sem).wait()
            pltpu.make_async_copy(dst_h.at[pl.ds(0, CH)], dstb, sem).wait()

        def transform_gather(off, srcb, dstb, idxsb, idxdb, sdstb, xlb, xrb,
                             sg1, sg2):
            @pl.loop(0, CH // LN)
            def _(i):
                o = pl.multiple_of(i * LN, LN)
                dv = dstb[pl.ds(o, LN)]
                idxsb[pl.ds(o, LN)] = srcb[pl.ds(o, LN)] + off
                idxdb[pl.ds(o, LN)] = dv + off
                sdstb[pl.ds(o, LN)] = dv

            pltpu.async_copy(xl_h.at[idxsb], xlb, sg1)
            pltpu.async_copy(xr_h.at[idxdb], xrb, sg2)

        def compute_chunk(sdstb, idxsb, idxdb, xlb, xrb, sg1, sg2):
            pltpu.make_async_copy(xl_h.at[idxsb], xlb, sg1).wait()
            pltpu.make_async_copy(xr_h.at[idxdb], xrb, sg2).wait()

            @pl.loop(0, CH // LN)
            def _(i):
                o = pl.multiple_of(i * LN, LN)
                dstv = sdstb[pl.ds(o, LN)]
                for j in range(LN):
                    e = o + j
                    accv = zv
                    for c in range(D // LN):
                        sl = pl.ds(c * LN, LN)
                        z = xlb[e, sl] + xrb[e, sl]
                        z = jnp.maximum(z, 0.2 * z)
                        accv = accv + z * att_v[sl]
                    logit = jnp.sum(accv)
                    exv = jnp.exp(jnp.full((LN,), logit))
                    for c in range(D // LN):
                        sl = pl.ds(c * LN, LN)
                        xlb[e, sl] = exv * xlb[e, sl]
                    didx = jnp.full((LN,), dstv[j], jnp.int32)
                    plsc.addupdate_scatter(den_t, [didx], exv, mask=m0)

            pltpu.sync_copy(xlb, acc.at[sdstb], add=True)

        for p in range(H // NC):
            h = cid * (H // NC) + p
            off = h * NP

            @pl.loop(0, FB)
            def _(r):
                for c in range(D // LN):
                    xr_a[r, pl.ds(c * LN, LN)] = zv

            @pl.loop(0, rows_sc // FB)
            def _(kblk):
                pltpu.sync_copy(
                    xr_a.at[pl.ds(0, FB)],
                    acc.at[pl.ds(sid * rows_sc + kblk * FB, FB)],
                )

            zero_den()
            pltpu.sync_copy(att_h.at[pl.ds(h * D, D)], att_v)
            plsc.subcore_barrier()

            idx_load(0, src_a, dst_a, ia)
            idx_wait(src_a, dst_a, ia)
            transform_gather(off, src_a, dst_a, idxs_a, idxd_a, sdst_a,
                             xl_a, xr_a, s1a, s2a)
            idx_load(1, src_b, dst_b, ib)

            @pl.loop(0, n_chunks // 2)
            def _(t):
                c0 = t * 2
                idx_wait(src_b, dst_b, ib)
                transform_gather(off, src_b, dst_b, idxs_b, idxd_b, sdst_b,
                                 xl_b, xr_b, s1b, s2b)

                @pl.when(c0 + 2 < n_chunks)
                def _():
                    idx_load(c0 + 2, src_a, dst_a, ia)

                compute_chunk(sdst_a, idxs_a, idxd_a, xl_a, xr_a, s1a, s2a)

                @pl.when(c0 + 2 < n_chunks)
                def _():
                    idx_wait(src_a, dst_a, ia)
                    transform_gather(off, src_a, dst_a, idxs_a, idxd_a,
                                     sdst_a, xl_a, xr_a, s1a, s2a)

                @pl.when(c0 + 3 < n_chunks)
                def _():
                    idx_load(c0 + 3, src_b, dst_b, ib)

                compute_chunk(sdst_b, idxs_b, idxd_b, xl_b, xr_b, s1b, s2b)

            plsc.subcore_barrier()
            pltpu.sync_copy(den_t, dstg.at[cid * NS + sid])
            plsc.subcore_barrier()
            reduce_stage(dstg.at[pl.ds(cid * NS, NS)])

            @pl.loop(0, rows_sc // FB)
            def _(kblk):
                kF = pl.multiple_of(kblk * FB, FB)
                r0 = sid * rows_sc + kF
                pltpu.sync_copy(acc.at[pl.ds(r0, FB)], xr_a.at[pl.ds(0, FB)])

                @pl.loop(0, FB // LN)
                def _(i):
                    o = pl.multiple_of(i * LN, LN)
                    redv = red_v[pl.ds(kF + o, LN)]
                    for j in range(LN):
                        dv = jnp.full((LN,), redv[j] + 1e-16)
                        for c in range(D // LN):
                            sl = pl.ds(c * LN, LN)
                            xr_a[o + j, sl] = xr_a[o + j, sl] / dv

                pltpu.sync_copy(
                    xr_a.at[pl.ds(0, FB)], msg_out.at[pl.ds(off + r0, FB)]
                )

            plsc.subcore_barrier()

    return k(xlt, xrt, srcf, dstf, attf)


def _phase2_combine(nm, cnt, Wg, NP, H, D):
    BN = 512

    def body(a_ref, c_ref, wg_ref, o_ref):
        a = a_ref[...]
        og = jnp.mean(a, axis=0)
        hh = jnp.where(og > 0, og, jnp.exp(og) - 1.0)
        deg = c_ref[:, 0:1]
        dis = jnp.where(deg > 0, lax.rsqrt(deg), 0.0)
        o_ref[...] = jnp.dot(hh, wg_ref[...], preferred_element_type=jnp.float32) * dis

    return pl.pallas_call(
        body,
        grid=(NP // BN,),
        in_specs=[
            pl.BlockSpec((H, BN, D), lambda i: (0, i, 0)),
            pl.BlockSpec((BN, D), lambda i: (i, 0)),
            pl.BlockSpec((D, D), lambda i: (0, 0)),
        ],
        out_specs=pl.BlockSpec((BN, D), lambda i: (i, 0)),
        out_shape=jax.ShapeDtypeStruct((NP, D), jnp.float32),
    )(nm, cnt, Wg)


def _phase3_gcn(hgp, srcf, dstf, NP, EP, D):
    esc = EP // (NC * NS)
    n_chunks = esc // CH3
    rows_sc = NP // NS
    mesh = plsc.VectorSubcoreMesh(
        core_axis_name="c", subcore_axis_name="s", num_cores=NC, num_subcores=NS
    )

    @functools.partial(
        pl.kernel,
        mesh=mesh,
        out_type=jax.ShapeDtypeStruct((NC * NP, D), jnp.float32),
        scratch_types=[
            pltpu.VMEM((CH3,), jnp.int32),
            pltpu.VMEM((CH3,), jnp.int32),
            pltpu.VMEM((CH3,), jnp.int32),
            pltpu.VMEM((CH3,), jnp.int32),
            pltpu.VMEM((CH3, D), jnp.float32),
            pltpu.VMEM((CH3, D), jnp.float32),
            pltpu.VMEM_SHARED((NP, D), jnp.float32),
            pltpu.SemaphoreType.DMA,
            pltpu.SemaphoreType.DMA,
        ],
        compiler_params=_SC_PARAMS,
    )
    def k(hg_h, src_h, dst_h, out_h,
          src_a, src_b, dst_a, dst_b, rows_a, rows_b, acc2, sa, sb):
        cid = lax.axis_index("c")
        sid = lax.axis_index("s")
        zv = jnp.zeros((LN,), jnp.float32)

        @pl.loop(0, FB)
        def _(r):
            for c in range(D // LN):
                rows_a[r, pl.ds(c * LN, LN)] = zv

        @pl.loop(0, rows_sc // FB)
        def _(kblk):
            pltpu.sync_copy(
                rows_a.at[pl.ds(0, FB)],
                acc2.at[pl.ds(sid * rows_sc + kblk * FB, FB)],
            )
        plsc.subcore_barrier()

        def load_chunk(c, srcb, dstb, rowsb, sg):
            base = (cid * NS + sid) * esc + c * CH3
            pltpu.sync_copy(src_h.at[pl.ds(base, CH3)], srcb)
            pltpu.sync_copy(dst_h.at[pl.ds(base, CH3)], dstb)
            pltpu.async_copy(hg_h.at[srcb], rowsb, sg)

        def flush_chunk(srcb, dstb, rowsb, sg):
            pltpu.make_async_copy(hg_h.at[srcb], rowsb, sg).wait()
            pltpu.sync_copy(rowsb, acc2.at[dstb], add=True)

        load_chunk(0, src_a, dst_a, rows_a, sa)

        @pl.loop(0, n_chunks // 2)
        def _(t):
            c0 = t * 2
            load_chunk(c0 + 1, src_b, dst_b, rows_b, sb)
            flush_chunk(src_a, dst_a, rows_a, sa)

            @pl.when(c0 + 2 < n_chunks)
            def _():
                load_chunk(c0 + 2, src_a, dst_a, rows_a, sa)

            flush_chunk(src_b, dst_b, rows_b, sb)

        plsc.subcore_barrier()

        @pl.loop(0, rows_sc // FB)
        def _(kblk):
            r0 = sid * rows_sc + kblk * FB
            pltpu.sync_copy(
                acc2.at[pl.ds(r0, FB)], out_h.at[pl.ds(cid * NP + r0, FB)]
            )
        plsc.subcore_barrier()

    return k(hgp, srcf, dstf)


def _phase4_finish(acc2, cnt, NP, D):
    BN = 512

    def body(a2_ref, c_ref, o_ref):
        s = a2_ref[0] + a2_ref[1]
        deg = c_ref[:, 0:1]
        dis = jnp.where(deg > 0, lax.rsqrt(deg), 0.0)
        o_ref[...] = s * dis

    return pl.pallas_call(
        body,
        grid=(NP // BN,),
        in_specs=[
            pl.BlockSpec((2, BN, D), lambda i: (0, i, 0)),
            pl.BlockSpec((BN, D), lambda i: (i, 0)),
        ],
        out_specs=pl.BlockSpec((BN, D), lambda i: (i, 0)),
        out_shape=jax.ShapeDtypeStruct((NP, D), jnp.float32),
    )(acc2, cnt)


def kernel(x, edge_index, Wl, Wr, att, Wg):
    N, D = x.shape
    H = att.shape[0]
    E = edge_index.shape[1]
    NP = -(-N // 2048) * 2048
    EP1 = -(-E // (NS * CB)) * (NS * CB)
    EP3 = -(-E // (2 * NC * NS * CH3)) * (2 * NC * NS * CH3)
    EP = max(EP1, EP3)

    src = edge_index[0].astype(jnp.int32)
    dst = edge_index[1].astype(jnp.int32)
    pad = jnp.full((EP - E,), NP - 1, jnp.int32)
    srcf = jnp.concatenate([src, pad])
    dstf = jnp.concatenate([dst, pad])

    xp = jnp.zeros((NP, D), jnp.float32).at[:N].set(x)
    Wl4 = Wl.reshape(D, H, D).transpose(1, 0, 2)
    Wr4 = Wr.reshape(D, H, D).transpose(1, 0, 2)
    attf = att.reshape(H * D)

    xlt, xrt = _phase0_matmuls(xp, Wl4, Wr4, NP, H, D)
    xlt = xlt.reshape(H * NP, D)
    xrt = xrt.reshape(H * NP, D)

    nmf, cnt, _, _ = _phase1_gat(xlt, xrt, srcf, dstf, attf, NP, EP1, H, D)
    nm = nmf.reshape(H, NP, D)

    hgp = _phase2_combine(nm, cnt, Wg, NP, H, D)

    acc2f = _phase3_gcn(hgp, srcf, dstf, NP, EP3, D)
    acc2 = acc2f.reshape(NC, NP, D)

    out = _phase4_finish(acc2, cnt, NP, D)
    return out[:N]


# speedup vs baseline: 1.2691x; 1.0500x over previous
import functools

import jax
import jax.numpy as jnp
from jax import lax
from jax.experimental import pallas as pl
from jax.experimental.pallas import tpu as pltpu
from jax.experimental.pallas import tpu_sc as plsc

NC, NS, LN = 2, 16, 16
CH = 64
CB = 256
FB = 64
_SC_PARAMS = pltpu.CompilerParams(needs_layout_passes=False)


def _phase0_matmuls(xp, Wl4, Wr4, NP, H, D):
    BN = 1024

    def body(x_ref, wl_ref, wr_ref, ol_ref, or_ref):
        xb = x_ref[...]
        ol_ref[0] = jnp.dot(xb, wl_ref[0], preferred_element_type=jnp.float32)
        or_ref[0] = jnp.dot(xb, wr_ref[0], preferred_element_type=jnp.float32)

    return pl.pallas_call(
        body,
        grid=(H, NP // BN),
        in_specs=[
            pl.BlockSpec((BN, D), lambda h, i: (i, 0)),
            pl.BlockSpec((1, D, D), lambda h, i: (h, 0, 0)),
            pl.BlockSpec((1, D, D), lambda h, i: (h, 0, 0)),
        ],
        out_specs=[
            pl.BlockSpec((1, BN, D), lambda h, i: (h, i, 0)),
            pl.BlockSpec((1, BN, D), lambda h, i: (h, i, 0)),
        ],
        out_shape=[jax.ShapeDtypeStruct((H, NP, D), jnp.float32)] * 2,
    )(xp, Wl4, Wr4)


def _phase1_gat(xlt, xrt, srcf, dstf, attf, NP, EP, H, D):
    esc = EP // NS
    n_chunks = esc // CH
    rows_sc = NP // NS
    mesh = plsc.VectorSubcoreMesh(
        core_axis_name="c", subcore_axis_name="s", num_cores=NC, num_subcores=NS
    )

    @functools.partial(
        pl.kernel,
        mesh=mesh,
        out_type=[
            jax.ShapeDtypeStruct((H * NP, D), jnp.float32),
            jax.ShapeDtypeStruct((NP, D), jnp.float32),
            jax.ShapeDtypeStruct((NC * NS, NP), jnp.float32),
            jax.ShapeDtypeStruct((NS, NP), jnp.float32),
        ],
        scratch_types=[
            pltpu.VMEM((CH,), jnp.int32),
            pltpu.VMEM((CH,), jnp.int32),
            pltpu.VMEM((CH,), jnp.int32),
            pltpu.VMEM((CH,), jnp.int32),
            pltpu.VMEM((CH,), jnp.int32),
            pltpu.VMEM((CH,), jnp.int32),
            pltpu.VMEM((CH,), jnp.int32),
            pltpu.VMEM((CH,), jnp.int32),
            pltpu.VMEM((CH, D), jnp.float32),
            pltpu.VMEM((CH, D), jnp.float32),
            pltpu.VMEM((CH, D), jnp.float32),
            pltpu.VMEM((CH, D), jnp.float32),
            pltpu.VMEM((CH,), jnp.int32),
            pltpu.VMEM((CH,), jnp.int32),
            pltpu.VMEM((CB,), jnp.int32),
            pltpu.VMEM((D,), jnp.float32),
            pltpu.VMEM((NP,), jnp.float32),
            pltpu.VMEM((NS, D), jnp.float32),
            pltpu.VMEM((NP // NS,), jnp.float32),
            pltpu.VMEM_SHARED((NP, D), jnp.float32),
            pltpu.SemaphoreType.DMA,
            pltpu.SemaphoreType.DMA,
            pltpu.SemaphoreType.DMA,
            pltpu.SemaphoreType.DMA,
            pltpu.SemaphoreType.DMA,
            pltpu.SemaphoreType.DMA,
        ],
        compiler_params=_SC_PARAMS,
    )
    def k(xl_h, xr_h, src_h, dst_h, att_h, msg_out, cnt_out, dstg, cstg,
          src_a, src_b, dst_a, dst_b, idxs_a, idxs_b, idxd_a, idxd_b,
          xl_a, xl_b, xr_a, xr_b, sdst_a, sdst_b, cb_v, att_v,
          den_t, stg_v, red_v, acc,
          s1a, s2a, s1b, s2b, ia, ib):
        cid = lax.axis_index("c")
        sid = lax.axis_index("s")
        zv = jnp.zeros((LN,), jnp.float32)
        ones = jnp.full((LN,), 1.0, jnp.float32)
        m0 = lax.iota(jnp.int32, LN) == 0

        def zero_den():
            @pl.loop(0, NP // LN)
            def _(i):
                o = pl.multiple_of(i * LN, LN)
                den_t[pl.ds(o, LN)] = zv

        def reduce_stage(stg):
            @pl.loop(0, rows_sc // D)
            def _(t):
                tD = pl.multiple_of(t * D, D)
                pltpu.sync_copy(
                    stg.at[:, pl.ds(sid * rows_sc + tD, D)], stg_v
                )

                @pl.loop(0, D // LN)
                def _(i):
                    o = pl.multiple_of(i * LN, LN)
                    sl = pl.ds(o, LN)
                    tv = zv
                    for s in range(NS):
                        tv = tv + stg_v[s, sl]
                    red_v[pl.ds(tD + o, LN)] = tv

        @pl.when(cid == 0)
        def _():
            zero_den()

            @pl.loop(0, esc // CB)
            def _(q):
                pltpu.sync_copy(dst_h.at[pl.ds(sid * esc + q * CB, CB)], cb_v)

                @pl.loop(0, CB // LN)
                def _(i):
                    o = pl.multiple_of(i * LN, LN)
                    dstv = cb_v[pl.ds(o, LN)]
                    for j in range(LN):
                        didx = jnp.full((LN,), dstv[j], jnp.int32)
                        plsc.addupdate_scatter(den_t, [didx], ones, mask=m0)

            pltpu.sync_copy(den_t, cstg.at[sid])
            plsc.subcore_barrier()
            reduce_stage(cstg)

            @pl.loop(0, rows_sc // FB)
            def _(kblk):
                kF = pl.multiple_of(kblk * FB, FB)

                @pl.loop(0, FB // LN)
                def _(i):
                    o = pl.multiple_of(i * LN, LN)
                    redv = red_v[pl.ds(kF + o, LN)]
                    for j in range(LN):
                        cv = jnp.full((LN,), redv[j])
                        for c in range(D // LN):
                            xr_a[o + j, pl.ds(c * LN, LN)] = cv

                pltpu.sync_copy(
                    xr_a.at[pl.ds(0, FB)],
                    cnt_out.at[pl.ds(sid * rows_sc + kF, FB)],
                )

        def idx_load(c, srcb, dstb, sem):
            base = sid * esc + c * CH
            pltpu.async_copy(src_h.at[pl.ds(base, CH)], srcb, sem)
            pltpu.async_copy(dst_h.at[pl.ds(base, CH)], dstb, sem)

        def idx_wait(srcb, dstb, sem):
            pltpu.make_async_copy(src_h.at[pl.ds(0, CH)], srcb, sem).wait()
            pltpu.make_async_copy(dst_h.at[pl.ds(0, CH)], dstb, sem).wait()

        def transform_gather(off, srcb, dstb, idxsb, idxdb, sdstb, xlb, xrb,
                             sg1, sg2):
            @pl.loop(0, CH // LN)
            def _(i):
                o = pl.multiple_of(i * LN, LN)
                dv = dstb[pl.ds(o, LN)]
                idxsb[pl.ds(o, LN)] = srcb[pl.ds(o, LN)] + off
                idxdb[pl.ds(o, LN)] = dv + off
                sdstb[pl.ds(o, LN)] = dv

            pltpu.async_copy(xl_h.at[idxsb], xlb, sg1)
            pltpu.async_copy(xr_h.at[idxdb], xrb, sg2)

        def compute_chunk(sdstb, idxsb, idxdb, xlb, xrb, sg1, sg2):
            pltpu.make_async_copy(xl_h.at[idxsb], xlb, sg1).wait()
            pltpu.make_async_copy(xr_h.at[idxdb], xrb, sg2).wait()

            @pl.loop(0, CH // LN)
            def _(i):
                o = pl.multiple_of(i * LN, LN)
                dstv = sdstb[pl.ds(o, LN)]
                for j in range(LN):
                    e = o + j
                    accv = zv
                    for c in range(D // LN):
                        sl = pl.ds(c * LN, LN)
                        z = xlb[e, sl] + xrb[e, sl]
                        z = jnp.maximum(z, 0.2 * z)
                        accv = accv + z * att_v[sl]
                    logit = jnp.sum(accv)
                    exv = jnp.exp(jnp.full((LN,), logit))
                    for c in range(D // LN):
                        sl = pl.ds(c * LN, LN)
                        xlb[e, sl] = exv * xlb[e, sl]
                    didx = jnp.full((LN,), dstv[j], jnp.int32)
                    plsc.addupdate_scatter(den_t, [didx], exv, mask=m0)

            pltpu.sync_copy(xlb, acc.at[sdstb], add=True)

        for p in range(H // NC):
            h = cid * (H // NC) + p
            off = h * NP

            @pl.loop(0, FB)
            def _(r):
                for c in range(D // LN):
                    xr_a[r, pl.ds(c * LN, LN)] = zv

            @pl.loop(0, rows_sc // FB)
            def _(kblk):
                pltpu.sync_copy(
                    xr_a.at[pl.ds(0, FB)],
                    acc.at[pl.ds(sid * rows_sc + kblk * FB, FB)],
                )

            zero_den()
            pltpu.sync_copy(att_h.at[pl.ds(h * D, D)], att_v)
            plsc.subcore_barrier()

            idx_load(0, src_a, dst_a, ia)
            idx_wait(src_a, dst_a, ia)
            transform_gather(off, src_a, dst_a, idxs_a, idxd_a, sdst_a,
                             xl_a, xr_a, s1a, s2a)
            idx_load(1, src_b, dst_b, ib)

            @pl.loop(0, n_chunks // 2)
            def _(t):
                c0 = t * 2
                idx_wait(src_b, dst_b, ib)
                transform_gather(off, src_b, dst_b, idxs_b, idxd_b, sdst_b,
                                 xl_b, xr_b, s1b, s2b)

                @pl.when(c0 + 2 < n_chunks)
                def _():
                    idx_load(c0 + 2, src_a, dst_a, ia)

                compute_chunk(sdst_a, idxs_a, idxd_a, xl_a, xr_a, s1a, s2a)

                @pl.when(c0 + 2 < n_chunks)
                def _():
                    idx_wait(src_a, dst_a, ia)
                    transform_gather(off, src_a, dst_a, idxs_a, idxd_a,
                                     sdst_a, xl_a, xr_a, s1a, s2a)

                @pl.when(c0 + 3 < n_chunks)
                def _():
                    idx_load(c0 + 3, src_b, dst_b, ib)

                compute_chunk(sdst_b, idxs_b, idxd_b, xl_b, xr_b, s1b, s2b)

            plsc.subcore_barrier()
            pltpu.sync_copy(den_t, dstg.at[cid * NS + sid])
            plsc.subcore_barrier()
            reduce_stage(dstg.at[pl.ds(cid * NS, NS)])

            @pl.loop(0, rows_sc // FB)
            def _(kblk):
                kF = pl.multiple_of(kblk * FB, FB)
                r0 = sid * rows_sc + kF
                pltpu.sync_copy(acc.at[pl.ds(r0, FB)], xr_a.at[pl.ds(0, FB)])

                @pl.loop(0, FB // LN)
                def _(i):
                    o = pl.multiple_of(i * LN, LN)
                    redv = red_v[pl.ds(kF + o, LN)]
                    for j in range(LN):
                        dv = jnp.full((LN,), redv[j] + 1e-16)
                        for c in range(D // LN):
                            sl = pl.ds(c * LN, LN)
                            xr_a[o + j, sl] = xr_a[o + j, sl] / dv

                pltpu.sync_copy(
                    xr_a.at[pl.ds(0, FB)], msg_out.at[pl.ds(off + r0, FB)]
                )

            plsc.subcore_barrier()

    return k(xlt, xrt, srcf, dstf, attf)


def _phase2_combine(nm, cnt, Wg, NP, H, D):
    BN = 512

    def body(a_ref, c_ref, wg_ref, o_ref):
        a = a_ref[...]
        og = jnp.mean(a, axis=0)
        hh = jnp.where(og > 0, og, jnp.exp(og) - 1.0)
        deg = c_ref[:, 0:1]
        dis = jnp.where(deg > 0, lax.rsqrt(deg), 0.0)
        o_ref[...] = jnp.dot(hh, wg_ref[...], preferred_element_type=jnp.float32) * dis

    return pl.pallas_call(
        body,
        grid=(NP // BN,),
        in_specs=[
            pl.BlockSpec((H, BN, D), lambda i: (0, i, 0)),
            pl.BlockSpec((BN, D), lambda i: (i, 0)),
            pl.BlockSpec((D, D), lambda i: (0, 0)),
        ],
        out_specs=pl.BlockSpec((BN, D), lambda i: (i, 0)),
        out_shape=jax.ShapeDtypeStruct((NP, D), jnp.float32),
    )(nm, cnt, Wg)


def _phase3_gcn(hgp, srcf, dstf, NP, EP, D):
    esc = EP // (NC * NS)
    n_chunks = esc // CH
    rows_sc = NP // NS
    mesh = plsc.VectorSubcoreMesh(
        core_axis_name="c", subcore_axis_name="s", num_cores=NC, num_subcores=NS
    )

    @functools.partial(
        pl.kernel,
        mesh=mesh,
        out_type=jax.ShapeDtypeStruct((NC * NP, D), jnp.float32),
        scratch_types=[
            pltpu.VMEM((CH,), jnp.int32),
            pltpu.VMEM((CH,), jnp.int32),
            pltpu.VMEM((CH,), jnp.int32),
            pltpu.VMEM((CH,), jnp.int32),
            pltpu.VMEM((CH, D), jnp.float32),
            pltpu.VMEM((CH, D), jnp.float32),
            pltpu.VMEM_SHARED((NP, D), jnp.float32),
            pltpu.SemaphoreType.DMA,
            pltpu.SemaphoreType.DMA,
        ],
        compiler_params=_SC_PARAMS,
    )
    def k(hg_h, src_h, dst_h, out_h,
          src_a, src_b, dst_a, dst_b, rows_a, rows_b, acc2, sa, sb):
        cid = lax.axis_index("c")
        sid = lax.axis_index("s")
        zv = jnp.zeros((LN,), jnp.float32)

        @pl.loop(0, FB)
        def _(r):
            for c in range(D // LN):
                rows_a[r, pl.ds(c * LN, LN)] = zv

        @pl.loop(0, rows_sc // FB)
        def _(kblk):
            pltpu.sync_copy(
                rows_a.at[pl.ds(0, FB)],
                acc2.at[pl.ds(sid * rows_sc + kblk * FB, FB)],
            )
        plsc.subcore_barrier()

        def load_chunk(c, srcb, dstb, rowsb, sg):
            base = (cid * NS + sid) * esc + c * CH
            pltpu.sync_copy(src_h.at[pl.ds(base, CH)], srcb)
            pltpu.sync_copy(dst_h.at[pl.ds(base, CH)], dstb)
            pltpu.async_copy(hg_h.at[srcb], rowsb, sg)

        def flush_chunk(srcb, dstb, rowsb, sg):
            pltpu.make_async_copy(hg_h.at[srcb], rowsb, sg).wait()
            pltpu.sync_copy(rowsb, acc2.at[dstb], add=True)

        load_chunk(0, src_a, dst_a, rows_a, sa)

        @pl.loop(0, n_chunks // 2)
        def _(t):
            c0 = t * 2
            load_chunk(c0 + 1, src_b, dst_b, rows_b, sb)
            flush_chunk(src_a, dst_a, rows_a, sa)

            @pl.when(c0 + 2 < n_chunks)
            def _():
                load_chunk(c0 + 2, src_a, dst_a, rows_a, sa)

            flush_chunk(src_b, dst_b, rows_b, sb)

        plsc.subcore_barrier()

        @pl.loop(0, rows_sc // FB)
        def _(kblk):
            r0 = sid * rows_sc + kblk * FB
            pltpu.sync_copy(
                acc2.at[pl.ds(r0, FB)], out_h.at[pl.ds(cid * NP + r0, FB)]
            )
        plsc.subcore_barrier()

    return k(hgp, srcf, dstf)


def _phase4_finish(acc2, cnt, NP, D):
    BN = 512

    def body(a2_ref, c_ref, o_ref):
        s = a2_ref[0] + a2_ref[1]
        deg = c_ref[:, 0:1]
        dis = jnp.where(deg > 0, lax.rsqrt(deg), 0.0)
        o_ref[...] = s * dis

    return pl.pallas_call(
        body,
        grid=(NP // BN,),
        in_specs=[
            pl.BlockSpec((2, BN, D), lambda i: (0, i, 0)),
            pl.BlockSpec((BN, D), lambda i: (i, 0)),
        ],
        out_specs=pl.BlockSpec((BN, D), lambda i: (i, 0)),
        out_shape=jax.ShapeDtypeStruct((NP, D), jnp.float32),
    )(acc2, cnt)


def kernel(x, edge_index, Wl, Wr, att, Wg):
    N, D = x.shape
    H = att.shape[0]
    E = edge_index.shape[1]
    NP = -(-N // 2048) * 2048
    EP = -(-E // (2 * NC * NS * CH)) * (2 * NC * NS * CH)

    src = edge_index[0].astype(jnp.int32)
    dst = edge_index[1].astype(jnp.int32)
    pad = jnp.full((EP - E,), NP - 1, jnp.int32)
    srcf = jnp.concatenate([src, pad])
    dstf = jnp.concatenate([dst, pad])

    xp = jnp.zeros((NP, D), jnp.float32).at[:N].set(x)
    Wl4 = Wl.reshape(D, H, D).transpose(1, 0, 2)
    Wr4 = Wr.reshape(D, H, D).transpose(1, 0, 2)
    attf = att.reshape(H * D)

    xlt, xrt = _phase0_matmuls(xp, Wl4, Wr4, NP, H, D)
    xlt = xlt.reshape(H * NP, D)
    xrt = xrt.reshape(H * NP, D)

    nmf, cnt, _, _ = _phase1_gat(xlt, xrt, srcf, dstf, attf, NP, EP, H, D)
    nm = nmf.reshape(H, NP, D)

    hgp = _phase2_combine(nm, cnt, Wg, NP, H, D)

    acc2f = _phase3_gcn(hgp, srcf, dstf, NP, EP, D)
    acc2 = acc2f.reshape(NC, NP, D)

    out = _phase4_finish(acc2, cnt, NP, D)
    return out[:N]
